# baseline (speedup 1.0000x reference)
import functools

import jax
import jax.numpy as jnp
from jax import lax
from jax.experimental import pallas as pl
from jax.experimental.pallas import tpu as pltpu
from jax.experimental.pallas import tpu_sc as plsc

_K = 1024
_D = 16384
_B = 512
_BK = 128
_P1 = _K // _BK
_BD = 512
_P2 = _D // _BD
_COMMIT = 0.25


def _vq_kernel(xt_ref, w1_ref, w2_ref, idx_ref, qt_ref, loss_ref, ppl_ref,
               rowsum_ref, minval_ref, minidx_ref, enct_ref):
    k = pl.program_id(0)

    @pl.when(k == 0)
    def _init():
        rowsum_ref[...] = jnp.sum(xt_ref[...] ** 2, axis=0, keepdims=True)
        minval_ref[...] = jnp.full((1, _B), jnp.inf, jnp.float32)
        minidx_ref[...] = jnp.zeros((1, _B), jnp.int32)

    @pl.when(k < _P1)
    def _phase1():
        w = w1_ref[...]
        wsum = jnp.sum(w ** 2, axis=1, keepdims=True)
        mm = lax.dot_general(w, xt_ref[...], (((1,), (0,)), ((), ())),
                             preferred_element_type=jnp.float32)
        d = (rowsum_ref[...] + wsum) - 2.0 * mm

        blkmin = jnp.min(d, axis=0, keepdims=True)
        rows = lax.broadcasted_iota(jnp.int32, d.shape, 0)
        blkarg = jnp.min(jnp.where(d == blkmin, rows, _K), axis=0,
                         keepdims=True) + k * _BK
        better = blkmin < minval_ref[...]
        minidx_ref[...] = jnp.where(better, blkarg, minidx_ref[...])
        minval_ref[...] = jnp.where(better, blkmin, minval_ref[...])

        @pl.when(k == _P1 - 1)
        def _finish():
            idx = minidx_ref[...]
            idx_ref[...] = idx
            enct = (lax.broadcasted_iota(jnp.int32, (_K, _B), 0) == idx
                    ).astype(jnp.float32)
            enct_ref[...] = enct
            loss_ref[0, 0] = jnp.sum(minval_ref[...]) * (
                (1.0 + _COMMIT) / (_B * _D))
            p = jnp.sum(enct, axis=1) * (1.0 / _B)
            ppl_ref[0, 0] = jnp.exp(-jnp.sum(p * jnp.log(p + 1e-10)))

    @pl.when(k >= _P1)
    def _phase2():
        qt_ref[...] = lax.dot_general(w2_ref[...], enct_ref[...],
                                      (((0,), (0,)), ((), ())),
                                      preferred_element_type=jnp.float32)


def _vq_tc(xt, W):
    return pl.pallas_call(
        _vq_kernel,
        grid=(_P1 + _P2,),
        in_specs=[
            pl.BlockSpec((_D, _B), lambda k: (0, 0)),
            pl.BlockSpec((_BK, _D), lambda k: (jnp.minimum(k, _P1 - 1), 0)),
            pl.BlockSpec((_K, _BD), lambda k: (0, jnp.maximum(k - _P1, 0))),
        ],
        out_specs=[
            pl.BlockSpec((1, _B), lambda k: (0, 0)),
            pl.BlockSpec((_BD, _B), lambda k: (jnp.maximum(k - _P1, 0), 0)),
            pl.BlockSpec(memory_space=pltpu.SMEM),
            pl.BlockSpec(memory_space=pltpu.SMEM),
        ],
        out_shape=[
            jax.ShapeDtypeStruct((1, _B), jnp.int32),
            jax.ShapeDtypeStruct((_D, _B), jnp.float32),
            jax.ShapeDtypeStruct((1, 1), jnp.float32),
            jax.ShapeDtypeStruct((1, 1), jnp.float32),
        ],
        scratch_shapes=[
            pltpu.VMEM((1, _B), jnp.float32),
            pltpu.VMEM((1, _B), jnp.float32),
            pltpu.VMEM((1, _B), jnp.int32),
            pltpu.VMEM((_K, _B), jnp.float32),
        ],
    )(xt, W, W)


_NW = 32
_RPW = _B // _NW


def _sc_onehot(idx):
    mesh = plsc.VectorSubcoreMesh(core_axis_name="c", subcore_axis_name="s")

    @functools.partial(
        pl.kernel,
        mesh=mesh,
        compiler_params=pltpu.CompilerParams(needs_layout_passes=False),
        out_type=jax.ShapeDtypeStruct((_B, _K), jnp.float32),
        scratch_types=[
            pltpu.VMEM((_RPW,), jnp.int32),
            pltpu.VMEM((_RPW, _K), jnp.float32),
        ],
    )
    def enc_k(idx_hbm, out_hbm, idx_v, buf):
        wid = lax.axis_index("s") * 2 + lax.axis_index("c")
        base = wid * _RPW
        pltpu.sync_copy(idx_hbm.at[pl.ds(base, _RPW)], idx_v)
        zeros = jnp.zeros((16,), jnp.float32)

        @plsc.parallel_loop(0, _K // 16, 1, unroll=4)
        def zfill(g):
            for r in range(_RPW):
                buf[r, pl.ds(g * 16, 16)] = zeros

        lanes = lax.iota(jnp.int32, 16)
        cols = plsc.load_gather(idx_v, [lanes])
        plsc.store_scatter(buf, [lanes, cols], zeros + 1.0)
        pltpu.sync_copy(buf, out_hbm.at[pl.ds(base, _RPW)])

    return enc_k(idx)


def kernel(inputs, W):
    input_shape = inputs.shape
    xt = inputs.reshape(_B, _D).T
    idx, qt, loss, ppl = _vq_tc(xt, W)
    encodings = _sc_onehot(idx.reshape(_B))
    quantized = qt.T.reshape(input_shape)
    return (loss.reshape(()), quantized, ppl.reshape(()), encodings)


# speedup vs baseline: 1.1366x; 1.1366x over previous
import functools

import jax
import jax.numpy as jnp
from jax import lax
from jax.experimental import pallas as pl
from jax.experimental.pallas import tpu as pltpu
from jax.experimental.pallas import tpu_sc as plsc

_K = 1024
_D = 16384
_B = 512
_BK = 128
_BD = 1024
_COMMIT = 0.25


def _distance_argmin_kernel(xt_ref, w_ref, idx_ref, enct_ref, loss_ref,
                            ppl_ref, rowsum_ref, minval_ref, minidx_ref):
    k = pl.program_id(0)

    @pl.when(k == 0)
    def _init():
        rowsum_ref[...] = jnp.sum(xt_ref[...] ** 2, axis=0, keepdims=True)
        minval_ref[...] = jnp.full((1, _B), jnp.inf, jnp.float32)
        minidx_ref[...] = jnp.zeros((1, _B), jnp.int32)

    w = w_ref[...]
    wsum = jnp.sum(w ** 2, axis=1, keepdims=True)
    mm = lax.dot_general(w, xt_ref[...], (((1,), (0,)), ((), ())),
                         preferred_element_type=jnp.float32)
    d = (rowsum_ref[...] + wsum) - 2.0 * mm

    blkmin = jnp.min(d, axis=0, keepdims=True)
    rows = lax.broadcasted_iota(jnp.int32, d.shape, 0)
    blkarg = jnp.min(jnp.where(d == blkmin, rows, _K), axis=0,
                     keepdims=True) + k * _BK
    better = blkmin < minval_ref[...]
    minidx_ref[...] = jnp.where(better, blkarg, minidx_ref[...])
    minval_ref[...] = jnp.where(better, blkmin, minval_ref[...])

    @pl.when(k == pl.num_programs(0) - 1)
    def _finish():
        idx = minidx_ref[...]
        idx_ref[...] = idx
        enct = (lax.broadcasted_iota(jnp.int32, (_K, _B), 0) == idx
                ).astype(jnp.float32)
        enct_ref[...] = enct
        loss_ref[0, 0] = jnp.sum(minval_ref[...]) * (
            (1.0 + _COMMIT) / (_B * _D))
        p = jnp.sum(enct, axis=1) * (1.0 / _B)
        ppl_ref[0, 0] = jnp.exp(-jnp.sum(p * jnp.log(p + 1e-10)))


def _distances_argmin(xt, W):
    grid = _K // _BK
    return pl.pallas_call(
        _distance_argmin_kernel,
        grid=(grid,),
        in_specs=[
            pl.BlockSpec((_D, _B), lambda k: (0, 0)),
            pl.BlockSpec((_BK, _D), lambda k: (k, 0)),
        ],
        out_specs=[
            pl.BlockSpec((1, _B), lambda k: (0, 0)),
            pl.BlockSpec((_K, _B), lambda k: (0, 0)),
            pl.BlockSpec(memory_space=pltpu.SMEM),
            pl.BlockSpec(memory_space=pltpu.SMEM),
        ],
        out_shape=[
            jax.ShapeDtypeStruct((1, _B), jnp.int32),
            jax.ShapeDtypeStruct((_K, _B), jnp.float32),
            jax.ShapeDtypeStruct((1, 1), jnp.float32),
            jax.ShapeDtypeStruct((1, 1), jnp.float32),
        ],
        scratch_shapes=[
            pltpu.VMEM((1, _B), jnp.float32),
            pltpu.VMEM((1, _B), jnp.float32),
            pltpu.VMEM((1, _B), jnp.int32),
        ],
    )(xt, W)


def _select_kernel(w_ref, enct_ref, qt_ref):
    qt_ref[...] = lax.dot_general(w_ref[...], enct_ref[...],
                                  (((0,), (0,)), ((), ())),
                                  preferred_element_type=jnp.float32)


def _select_rows_t(W, enct):
    grid = _D // _BD
    return pl.pallas_call(
        _select_kernel,
        grid=(grid,),
        in_specs=[
            pl.BlockSpec((_K, _BD), lambda j: (0, j)),
            pl.BlockSpec((_K, _B), lambda j: (0, 0)),
        ],
        out_specs=pl.BlockSpec((_BD, _B), lambda j: (j, 0)),
        out_shape=jax.ShapeDtypeStruct((_D, _B), jnp.float32),
    )(W, enct)


_NW = 32
_RPW = _B // _NW


def _sc_onehot(idx):
    mesh = plsc.VectorSubcoreMesh(core_axis_name="c", subcore_axis_name="s")

    @functools.partial(
        pl.kernel,
        mesh=mesh,
        compiler_params=pltpu.CompilerParams(needs_layout_passes=False),
        out_type=jax.ShapeDtypeStruct((_B, _K), jnp.float32),
        scratch_types=[
            pltpu.VMEM((_RPW,), jnp.int32),
            pltpu.VMEM((_RPW, _K), jnp.float32),
        ],
    )
    def enc_k(idx_hbm, out_hbm, idx_v, buf):
        wid = lax.axis_index("s") * 2 + lax.axis_index("c")
        base = wid * _RPW
        pltpu.sync_copy(idx_hbm.at[pl.ds(base, _RPW)], idx_v)
        zeros = jnp.zeros((16,), jnp.float32)

        @plsc.parallel_loop(0, _K // 16, 1, unroll=4)
        def zfill(g):
            for r in range(_RPW):
                buf[r, pl.ds(g * 16, 16)] = zeros

        lanes = lax.iota(jnp.int32, 16)
        cols = plsc.load_gather(idx_v, [lanes])
        plsc.store_scatter(buf, [lanes, cols], zeros + 1.0)
        pltpu.sync_copy(buf, out_hbm.at[pl.ds(base, _RPW)])

    return enc_k(idx)


def kernel(inputs, W):
    input_shape = inputs.shape
    xt = inputs.reshape(_B, _D).T
    idx, enct, loss, ppl = _distances_argmin(xt, W)
    qt = _select_rows_t(W, enct)
    encodings = _sc_onehot(idx.reshape(_B))
    quantized = qt.T.reshape(input_shape)
    return (loss.reshape(()), quantized, ppl.reshape(()), encodings)


# speedup vs baseline: 1.1464x; 1.0086x over previous
import functools

import jax
import jax.numpy as jnp
from jax import lax
from jax.experimental import pallas as pl
from jax.experimental.pallas import tpu as pltpu
from jax.experimental.pallas import tpu_sc as plsc

_K = 1024
_D = 16384
_B = 512
_BK = 128
_BD = 2048
_COMMIT = 0.25


def _distance_argmin_kernel(xt_ref, w_ref, idx_ref, enct_ref, loss_ref,
                            ppl_ref, rowsum_ref, minval_ref, minidx_ref):
    k = pl.program_id(0)

    @pl.when(k == 0)
    def _init():
        rowsum_ref[...] = jnp.sum(xt_ref[...] ** 2, axis=0, keepdims=True)
        minval_ref[...] = jnp.full((1, _B), jnp.inf, jnp.float32)
        minidx_ref[...] = jnp.zeros((1, _B), jnp.int32)

    w = w_ref[...]
    wsum = jnp.sum(w ** 2, axis=1, keepdims=True)
    mm = lax.dot_general(w, xt_ref[...], (((1,), (0,)), ((), ())),
                         preferred_element_type=jnp.float32)
    d = (rowsum_ref[...] + wsum) - 2.0 * mm

    blkmin = jnp.min(d, axis=0, keepdims=True)
    rows = lax.broadcasted_iota(jnp.int32, d.shape, 0)
    blkarg = jnp.min(jnp.where(d == blkmin, rows, _K), axis=0,
                     keepdims=True) + k * _BK
    better = blkmin < minval_ref[...]
    minidx_ref[...] = jnp.where(better, blkarg, minidx_ref[...])
    minval_ref[...] = jnp.where(better, blkmin, minval_ref[...])

    @pl.when(k == pl.num_programs(0) - 1)
    def _finish():
        idx = minidx_ref[...]
        idx_ref[...] = idx
        enct = (lax.broadcasted_iota(jnp.int32, (_K, _B), 0) == idx
                ).astype(jnp.float32)
        enct_ref[...] = enct
        loss_ref[0, 0] = jnp.sum(minval_ref[...]) * (
            (1.0 + _COMMIT) / (_B * _D))
        p = jnp.sum(enct, axis=1) * (1.0 / _B)
        ppl_ref[0, 0] = jnp.exp(-jnp.sum(p * jnp.log(p + 1e-10)))


def _distances_argmin(xt, W):
    grid = _K // _BK
    return pl.pallas_call(
        _distance_argmin_kernel,
        grid=(grid,),
        in_specs=[
            pl.BlockSpec((_D, _B), lambda k: (0, 0)),
            pl.BlockSpec((_BK, _D), lambda k: (k, 0)),
        ],
        out_specs=[
            pl.BlockSpec((1, _B), lambda k: (0, 0)),
            pl.BlockSpec((_K, _B), lambda k: (0, 0)),
            pl.BlockSpec(memory_space=pltpu.SMEM),
            pl.BlockSpec(memory_space=pltpu.SMEM),
        ],
        out_shape=[
            jax.ShapeDtypeStruct((1, _B), jnp.int32),
            jax.ShapeDtypeStruct((_K, _B), jnp.float32),
            jax.ShapeDtypeStruct((1, 1), jnp.float32),
            jax.ShapeDtypeStruct((1, 1), jnp.float32),
        ],
        scratch_shapes=[
            pltpu.VMEM((1, _B), jnp.float32),
            pltpu.VMEM((1, _B), jnp.float32),
            pltpu.VMEM((1, _B), jnp.int32),
        ],
    )(xt, W)


def _select_kernel(w_ref, enct_ref, qt_ref):
    qt_ref[...] = lax.dot_general(w_ref[...], enct_ref[...],
                                  (((0,), (0,)), ((), ())),
                                  preferred_element_type=jnp.float32)


def _select_rows_t(W, enct):
    grid = _D // _BD
    return pl.pallas_call(
        _select_kernel,
        grid=(grid,),
        in_specs=[
            pl.BlockSpec((_K, _BD), lambda j: (0, j)),
            pl.BlockSpec((_K, _B), lambda j: (0, 0)),
        ],
        out_specs=pl.BlockSpec((_BD, _B), lambda j: (j, 0)),
        out_shape=jax.ShapeDtypeStruct((_D, _B), jnp.float32),
    )(W, enct)


_NW = 32
_RPW = _B // _NW


def _sc_onehot(idx):
    mesh = plsc.VectorSubcoreMesh(core_axis_name="c", subcore_axis_name="s")

    @functools.partial(
        pl.kernel,
        mesh=mesh,
        compiler_params=pltpu.CompilerParams(needs_layout_passes=False),
        out_type=jax.ShapeDtypeStruct((_B, _K), jnp.float32),
        scratch_types=[
            pltpu.VMEM((_RPW,), jnp.int32),
            pltpu.VMEM((_RPW, _K), jnp.float32),
        ],
    )
    def enc_k(idx_hbm, out_hbm, idx_v, buf):
        wid = lax.axis_index("s") * 2 + lax.axis_index("c")
        base = wid * _RPW
        pltpu.sync_copy(idx_hbm.at[pl.ds(base, _RPW)], idx_v)
        zeros = jnp.zeros((16,), jnp.float32)

        @plsc.parallel_loop(0, _K // 16, 1, unroll=4)
        def zfill(g):
            for r in range(_RPW):
                buf[r, pl.ds(g * 16, 16)] = zeros

        lanes = lax.iota(jnp.int32, 16)
        cols = plsc.load_gather(idx_v, [lanes])
        plsc.store_scatter(buf, [lanes, cols], zeros + 1.0)
        pltpu.sync_copy(buf, out_hbm.at[pl.ds(base, _RPW)])

    return enc_k(idx)


def kernel(inputs, W):
    input_shape = inputs.shape
    xt = inputs.reshape(_B, _D).T
    idx, enct, loss, ppl = _distances_argmin(xt, W)
    qt = _select_rows_t(W, enct)
    encodings = _sc_onehot(idx.reshape(_B))
    quantized = qt.T.reshape(input_shape)
    return (loss.reshape(()), quantized, ppl.reshape(()), encodings)


# speedup vs baseline: 1.1754x; 1.0253x over previous
import functools

import jax
import jax.numpy as jnp
from jax import lax
from jax.experimental import pallas as pl
from jax.experimental.pallas import tpu as pltpu
from jax.experimental.pallas import tpu_sc as plsc

_K = 1024
_D = 16384
_B = 512
_BK = 128
_BD = 4096
_COMMIT = 0.25


def _distance_argmin_kernel(xt_ref, w_ref, idx_ref, enct_ref, loss_ref,
                            ppl_ref, rowsum_ref, minval_ref, minidx_ref):
    k = pl.program_id(0)

    @pl.when(k == 0)
    def _init():
        rowsum_ref[...] = jnp.sum(xt_ref[...] ** 2, axis=0, keepdims=True)
        minval_ref[...] = jnp.full((1, _B), jnp.inf, jnp.float32)
        minidx_ref[...] = jnp.zeros((1, _B), jnp.int32)

    w = w_ref[...]
    wsum = jnp.sum(w ** 2, axis=1, keepdims=True)
    mm = lax.dot_general(w, xt_ref[...], (((1,), (0,)), ((), ())),
                         preferred_element_type=jnp.float32)
    d = (rowsum_ref[...] + wsum) - 2.0 * mm

    blkmin = jnp.min(d, axis=0, keepdims=True)
    rows = lax.broadcasted_iota(jnp.int32, d.shape, 0)
    blkarg = jnp.min(jnp.where(d == blkmin, rows, _K), axis=0,
                     keepdims=True) + k * _BK
    better = blkmin < minval_ref[...]
    minidx_ref[...] = jnp.where(better, blkarg, minidx_ref[...])
    minval_ref[...] = jnp.where(better, blkmin, minval_ref[...])

    @pl.when(k == pl.num_programs(0) - 1)
    def _finish():
        idx = minidx_ref[...]
        idx_ref[...] = idx
        enct = (lax.broadcasted_iota(jnp.int32, (_K, _B), 0) == idx
                ).astype(jnp.float32)
        enct_ref[...] = enct
        loss_ref[0, 0] = jnp.sum(minval_ref[...]) * (
            (1.0 + _COMMIT) / (_B * _D))
        p = jnp.sum(enct, axis=1) * (1.0 / _B)
        ppl_ref[0, 0] = jnp.exp(-jnp.sum(p * jnp.log(p + 1e-10)))


def _distances_argmin(xt, W):
    grid = _K // _BK
    return pl.pallas_call(
        _distance_argmin_kernel,
        grid=(grid,),
        in_specs=[
            pl.BlockSpec((_D, _B), lambda k: (0, 0)),
            pl.BlockSpec((_BK, _D), lambda k: (k, 0)),
        ],
        out_specs=[
            pl.BlockSpec((1, _B), lambda k: (0, 0)),
            pl.BlockSpec((_K, _B), lambda k: (0, 0)),
            pl.BlockSpec(memory_space=pltpu.SMEM),
            pl.BlockSpec(memory_space=pltpu.SMEM),
        ],
        out_shape=[
            jax.ShapeDtypeStruct((1, _B), jnp.int32),
            jax.ShapeDtypeStruct((_K, _B), jnp.float32),
            jax.ShapeDtypeStruct((1, 1), jnp.float32),
            jax.ShapeDtypeStruct((1, 1), jnp.float32),
        ],
        scratch_shapes=[
            pltpu.VMEM((1, _B), jnp.float32),
            pltpu.VMEM((1, _B), jnp.float32),
            pltpu.VMEM((1, _B), jnp.int32),
        ],
    )(xt, W)


def _select_kernel(w_ref, enct_ref, qt_ref):
    qt_ref[...] = lax.dot_general(w_ref[...], enct_ref[...],
                                  (((0,), (0,)), ((), ())),
                                  preferred_element_type=jnp.float32)


def _select_rows_t(W, enct):
    grid = _D // _BD
    return pl.pallas_call(
        _select_kernel,
        grid=(grid,),
        in_specs=[
            pl.BlockSpec((_K, _BD), lambda j: (0, j)),
            pl.BlockSpec((_K, _B), lambda j: (0, 0)),
        ],
        out_specs=pl.BlockSpec((_BD, _B), lambda j: (j, 0)),
        out_shape=jax.ShapeDtypeStruct((_D, _B), jnp.float32),
    )(W, enct)


_NW = 32
_RPW = _B // _NW


def _sc_onehot(idx):
    mesh = plsc.VectorSubcoreMesh(core_axis_name="c", subcore_axis_name="s")

    @functools.partial(
        pl.kernel,
        mesh=mesh,
        compiler_params=pltpu.CompilerParams(needs_layout_passes=False),
        out_type=jax.ShapeDtypeStruct((_B, _K), jnp.float32),
        scratch_types=[
            pltpu.VMEM((_RPW,), jnp.int32),
            pltpu.VMEM((_RPW, _K), jnp.float32),
        ],
    )
    def enc_k(idx_hbm, out_hbm, idx_v, buf):
        wid = lax.axis_index("s") * 2 + lax.axis_index("c")
        base = wid * _RPW
        pltpu.sync_copy(idx_hbm.at[pl.ds(base, _RPW)], idx_v)
        zeros = jnp.zeros((16,), jnp.float32)

        @plsc.parallel_loop(0, _K // 16, 1, unroll=4)
        def zfill(g):
            for r in range(_RPW):
                buf[r, pl.ds(g * 16, 16)] = zeros

        lanes = lax.iota(jnp.int32, 16)
        cols = plsc.load_gather(idx_v, [lanes])
        plsc.store_scatter(buf, [lanes, cols], zeros + 1.0)
        pltpu.sync_copy(buf, out_hbm.at[pl.ds(base, _RPW)])

    return enc_k(idx)


def kernel(inputs, W):
    input_shape = inputs.shape
    xt = inputs.reshape(_B, _D).T
    idx, enct, loss, ppl = _distances_argmin(xt, W)
    qt = _select_rows_t(W, enct)
    encodings = _sc_onehot(idx.reshape(_B))
    quantized = qt.T.reshape(input_shape)
    return (loss.reshape(()), quantized, ppl.reshape(()), encodings)
